# baseline (speedup 1.0000x reference)
import functools

import jax
import jax.numpy as jnp
from jax import lax
from jax.experimental import pallas as pl
from jax.experimental.pallas import tpu as pltpu
from jax.experimental.pallas import tpu_sc as plsc

NUM_CORES = 2
NUM_SUBCORES = 16
LANES = 16
CHUNK = 32


def _build_sc_kernel(N, T, D):
    n_workers = NUM_CORES * NUM_SUBCORES
    rows_w = N // n_workers
    n_chunks = rows_w // CHUNK
    vecs_per_chunk = CHUNK * D // LANES
    vecs_per_row = D // LANES

    mesh = plsc.VectorSubcoreMesh(
        core_axis_name="c", subcore_axis_name="s",
        num_cores=NUM_CORES, num_subcores=NUM_SUBCORES)

    @functools.partial(
        pl.kernel,
        out_type=jax.ShapeDtypeStruct((N, D), jnp.float32),
        mesh=mesh,
        scratch_types=[
            pltpu.VMEM((rows_w,), jnp.int32),
            pltpu.VMEM((CHUNK, D), jnp.float32),
            pltpu.VMEM((CHUNK, D), jnp.float32),
            pltpu.SemaphoreType.DMA,
        ],
    )
    def sc_kernel(x_hbm, tok_hbm, pos_hbm, out_hbm, idx_v, gbuf, pbuf, sem):
        wid = lax.axis_index("s") * NUM_CORES + lax.axis_index("c")
        base = wid * rows_w
        t0 = lax.rem(base, T)
        pltpu.sync_copy(x_hbm.at[pl.ds(base, rows_w)], idx_v)

        def chunk_body(j, carry):
            r0 = j * CHUNK
            pltpu.sync_copy(pos_hbm.at[pl.ds(t0 + r0, CHUNK)], pbuf)
            pltpu.async_copy(
                tok_hbm.at[idx_v.at[pl.ds(r0, CHUNK)]], gbuf, sem).wait()

            def add_body(i, c):
                r = i // vecs_per_row
                col = (i % vecs_per_row) * LANES
                gbuf[r, pl.ds(col, LANES)] = (
                    gbuf[r, pl.ds(col, LANES)] + pbuf[r, pl.ds(col, LANES)])
                return c

            lax.fori_loop(0, vecs_per_chunk, add_body, 0)
            pltpu.sync_copy(gbuf, out_hbm.at[pl.ds(base + r0, CHUNK)])
            return carry

        lax.fori_loop(0, n_chunks, chunk_body, 0)

    return sc_kernel


def kernel(x, tok_emb, pos_emb):
    B, T = x.shape
    V, D = tok_emb.shape
    N = B * T
    sc_kernel = _build_sc_kernel(N, T, D)
    out = sc_kernel(x.reshape(N), tok_emb, pos_emb)
    return out.reshape(B, T, D)


# speedup vs baseline: 1.5944x; 1.5944x over previous
import functools

import jax
import jax.numpy as jnp
from jax import lax
from jax.experimental import pallas as pl
from jax.experimental.pallas import tpu as pltpu
from jax.experimental.pallas import tpu_sc as plsc

NUM_CORES = 2
NUM_SUBCORES = 16
LANES = 16
CHUNK = 32


def _build_sc_kernel(N, T, D):
    n_workers = NUM_CORES * NUM_SUBCORES
    rows_w = N // n_workers
    n_chunks = rows_w // CHUNK
    vecs_per_chunk = CHUNK * D // LANES
    vecs_per_row = D // LANES

    mesh = plsc.VectorSubcoreMesh(
        core_axis_name="c", subcore_axis_name="s",
        num_cores=NUM_CORES, num_subcores=NUM_SUBCORES)

    @functools.partial(
        pl.kernel,
        out_type=jax.ShapeDtypeStruct((N, D), jnp.float32),
        mesh=mesh,
        scratch_types=[
            pltpu.VMEM((rows_w,), jnp.int32),
            pltpu.VMEM((CHUNK, D), jnp.float32),
            pltpu.VMEM((CHUNK, D), jnp.float32),
            pltpu.SemaphoreType.DMA,
        ],
    )
    def sc_kernel(x_hbm, tok_hbm, pos_hbm, out_hbm, idx_v, gbuf, pbuf, sem):
        wid = lax.axis_index("s") * NUM_CORES + lax.axis_index("c")
        base = wid * rows_w
        t0 = lax.rem(base, T)
        pltpu.sync_copy(x_hbm.at[pl.ds(base, rows_w)], idx_v)

        def chunk_body(j, carry):
            r0 = j * CHUNK
            pltpu.sync_copy(pos_hbm.at[pl.ds(t0 + r0, CHUNK)], pbuf)
            pltpu.async_copy(
                tok_hbm.at[idx_v.at[pl.ds(r0, CHUNK)]], gbuf, sem).wait()

            def add_row(r, c):
                for col in range(vecs_per_row):
                    sl = pl.ds(col * LANES, LANES)
                    gbuf[r, sl] = gbuf[r, sl] + pbuf[r, sl]
                return c

            lax.fori_loop(0, CHUNK, add_row, 0)
            pltpu.sync_copy(gbuf, out_hbm.at[pl.ds(base + r0, CHUNK)])
            return carry

        lax.fori_loop(0, n_chunks, chunk_body, 0)

    return sc_kernel


def kernel(x, tok_emb, pos_emb):
    B, T = x.shape
    V, D = tok_emb.shape
    N = B * T
    sc_kernel = _build_sc_kernel(N, T, D)
    out = sc_kernel(x.reshape(N), tok_emb, pos_emb)
    return out.reshape(B, T, D)


# speedup vs baseline: 2.1081x; 1.3222x over previous
import functools

import jax
import jax.numpy as jnp
from jax import lax
from jax.experimental import pallas as pl
from jax.experimental.pallas import tpu as pltpu
from jax.experimental.pallas import tpu_sc as plsc

NUM_CORES = 2
NUM_SUBCORES = 16
LANES = 16
CHUNK = 16


def _build_sc_kernel(N, T, D):
    n_workers = NUM_CORES * NUM_SUBCORES
    rows_w = N // n_workers
    n_chunks = rows_w // CHUNK
    vecs_per_row = D // LANES

    mesh = plsc.VectorSubcoreMesh(
        core_axis_name="c", subcore_axis_name="s",
        num_cores=NUM_CORES, num_subcores=NUM_SUBCORES)

    @functools.partial(
        pl.kernel,
        out_type=jax.ShapeDtypeStruct((N, D), jnp.float32),
        mesh=mesh,
        scratch_types=[
            pltpu.VMEM((rows_w,), jnp.int32),
            pltpu.VMEM((2, CHUNK, D), jnp.float32),
            pltpu.VMEM((2, CHUNK, D), jnp.float32),
            pltpu.SemaphoreType.DMA,
            pltpu.SemaphoreType.DMA,
            pltpu.SemaphoreType.DMA,
            pltpu.SemaphoreType.DMA,
            pltpu.SemaphoreType.DMA,
            pltpu.SemaphoreType.DMA,
        ],
    )
    def sc_kernel(x_hbm, tok_hbm, pos_hbm, out_hbm, idx_v, gbuf, pbuf,
                  gsem0, gsem1, psem0, psem1, osem0, osem1):
        wid = lax.axis_index("s") * NUM_CORES + lax.axis_index("c")
        base = wid * rows_w
        t0 = lax.rem(base, T)
        gsems = (gsem0, gsem1)
        psems = (psem0, psem1)
        osems = (osem0, osem1)

        pltpu.sync_copy(x_hbm.at[pl.ds(base, rows_w)], idx_v)

        def fire_in(j, slot):
            r0 = j * CHUNK
            pltpu.async_copy(
                pos_hbm.at[pl.ds(t0 + r0, CHUNK)], pbuf.at[slot], psems[slot])
            pltpu.async_copy(
                tok_hbm.at[idx_v.at[pl.ds(r0, CHUNK)]], gbuf.at[slot],
                gsems[slot])

        fire_in(0, 0)

        @pl.loop(0, n_chunks, step=2)
        def pipeline(jj):
            for b in (0, 1):
                j = jj + b
                nxt = 1 - b

                @pl.when(j + 1 < n_chunks)
                def _():
                    @pl.when(j >= 1)
                    def _():
                        pltpu.make_async_copy(
                            gbuf.at[nxt],
                            out_hbm.at[pl.ds(base + (j - 1) * CHUNK, CHUNK)],
                            osems[nxt]).wait()
                    fire_in(j + 1, nxt)

                pltpu.make_async_copy(
                    pos_hbm.at[pl.ds(t0 + j * CHUNK, CHUNK)], pbuf.at[b],
                    psems[b]).wait()
                pltpu.make_async_copy(
                    tok_hbm.at[idx_v.at[pl.ds(j * CHUNK, CHUNK)]],
                    gbuf.at[b], gsems[b]).wait()

                def add_row(r, c):
                    for col in range(vecs_per_row):
                        sl = pl.ds(col * LANES, LANES)
                        gbuf[b, r, sl] = gbuf[b, r, sl] + pbuf[b, r, sl]
                    return c

                lax.fori_loop(0, CHUNK, add_row, 0)
                pltpu.async_copy(
                    gbuf.at[b], out_hbm.at[pl.ds(base + j * CHUNK, CHUNK)],
                    osems[b])

        pltpu.make_async_copy(
            gbuf.at[0], out_hbm.at[pl.ds(base + (n_chunks - 2) * CHUNK, CHUNK)],
            osems[0]).wait()
        pltpu.make_async_copy(
            gbuf.at[1], out_hbm.at[pl.ds(base + (n_chunks - 1) * CHUNK, CHUNK)],
            osems[1]).wait()

    return sc_kernel


def kernel(x, tok_emb, pos_emb):
    B, T = x.shape
    V, D = tok_emb.shape
    N = B * T
    sc_kernel = _build_sc_kernel(N, T, D)
    out = sc_kernel(x.reshape(N), tok_emb, pos_emb)
    return out.reshape(B, T, D)


# speedup vs baseline: 3.8052x; 1.8050x over previous
import functools

import jax
import jax.numpy as jnp
from jax import lax
from jax.experimental import pallas as pl
from jax.experimental.pallas import tpu as pltpu
from jax.experimental.pallas import tpu_sc as plsc

NUM_CORES = 2
NUM_SUBCORES = 16
LANES = 16
CHUNK = 8


def _build_sc_kernel(B, N, T, D):
    n_workers = NUM_CORES * NUM_SUBCORES
    t_w = T // n_workers
    n_chunks = t_w // CHUNK
    vecs_per_row = D // LANES

    mesh = plsc.VectorSubcoreMesh(
        core_axis_name="c", subcore_axis_name="s",
        num_cores=NUM_CORES, num_subcores=NUM_SUBCORES)

    @functools.partial(
        pl.kernel,
        out_type=jax.ShapeDtypeStruct((N, D), jnp.float32),
        mesh=mesh,
        scratch_types=[
            pltpu.VMEM((B * t_w,), jnp.int32),
            pltpu.VMEM((2, B, CHUNK, D), jnp.float32),
            pltpu.VMEM((2, CHUNK, D), jnp.float32),
            pltpu.SemaphoreType.DMA,
            pltpu.SemaphoreType.DMA,
            pltpu.SemaphoreType.DMA,
            pltpu.SemaphoreType.DMA,
            pltpu.SemaphoreType.DMA,
            pltpu.SemaphoreType.DMA,
        ],
    )
    def sc_kernel(x_hbm, tok_hbm, pos_hbm, out_hbm, idx_v, gbuf, pbuf,
                  gsem0, gsem1, psem0, psem1, osem0, osem1):
        wid = lax.axis_index("s") * NUM_CORES + lax.axis_index("c")
        base_t = wid * t_w
        gsems = (gsem0, gsem1)
        psems = (psem0, psem1)
        osems = (osem0, osem1)

        for b in range(B):
            pltpu.sync_copy(x_hbm.at[pl.ds(b * T + base_t, t_w)],
                            idx_v.at[pl.ds(b * t_w, t_w)])

        def fire_in(j, slot):
            t_off = j * CHUNK
            pltpu.async_copy(
                pos_hbm.at[pl.ds(base_t + t_off, CHUNK)], pbuf.at[slot],
                psems[slot])
            for b in range(B):
                pltpu.async_copy(
                    tok_hbm.at[idx_v.at[pl.ds(b * t_w + t_off, CHUNK)]],
                    gbuf.at[slot, b], gsems[slot])

        def wait_in(j, slot):
            t_off = j * CHUNK
            pltpu.make_async_copy(
                pos_hbm.at[pl.ds(base_t + t_off, CHUNK)], pbuf.at[slot],
                psems[slot]).wait()
            for b in range(B):
                pltpu.make_async_copy(
                    tok_hbm.at[idx_v.at[pl.ds(b * t_w + t_off, CHUNK)]],
                    gbuf.at[slot, b], gsems[slot]).wait()

        def fire_out(j, slot):
            t_off = j * CHUNK
            for b in range(B):
                pltpu.async_copy(
                    gbuf.at[slot, b],
                    out_hbm.at[pl.ds(b * T + base_t + t_off, CHUNK)],
                    osems[slot])

        def wait_out(j, slot):
            t_off = j * CHUNK
            for b in range(B):
                pltpu.make_async_copy(
                    gbuf.at[slot, b],
                    out_hbm.at[pl.ds(b * T + base_t + t_off, CHUNK)],
                    osems[slot]).wait()

        fire_in(0, 0)

        @pl.loop(0, n_chunks, step=2)
        def pipeline(jj):
            for sl in (0, 1):
                j = jj + sl
                nxt = 1 - sl

                @pl.when(j + 1 < n_chunks)
                def _():
                    @pl.when(j >= 1)
                    def _():
                        wait_out(j - 1, nxt)
                    fire_in(j + 1, nxt)

                wait_in(j, sl)

                def add_row(r, c):
                    for col in range(vecs_per_row):
                        vsl = pl.ds(col * LANES, LANES)
                        vp = pbuf[sl, r, vsl]
                        for b in range(B):
                            gbuf[sl, b, r, vsl] = gbuf[sl, b, r, vsl] + vp
                    return c

                lax.fori_loop(0, CHUNK, add_row, 0)
                fire_out(j, sl)

        wait_out(n_chunks - 2, 0)
        wait_out(n_chunks - 1, 1)

    return sc_kernel


def kernel(x, tok_emb, pos_emb):
    B, T = x.shape
    V, D = tok_emb.shape
    N = B * T
    sc_kernel = _build_sc_kernel(B, N, T, D)
    out = sc_kernel(x.reshape(N), tok_emb, pos_emb)
    return out.reshape(B, T, D)
